# Initial kernel scaffold; baseline (speedup 1.0000x reference)
import jax
import jax.numpy as jnp
from jax.experimental import pallas as pl


def kernel(lit_idx, clause_idx, prob_index, params):
    raise NotImplementedError("write your pallas kernel here")


# speedup vs baseline: 2.4877x; 2.4877x over previous
import functools

import jax
import jax.numpy as jnp
from jax import lax
from jax.experimental import pallas as pl
from jax.experimental.pallas import tpu as pltpu
from jax.experimental.pallas import tpu_sc as plsc

DIM = 128
N_VARS = 12800
N_LITS = 2 * N_VARS
N_CLAUSES = 51200
N_CELLS = 307200
N_PROBS = 64
N_ROUNDS = 4

BLK = 512
SLAB = 12800
N_DUMMY = 128
N_TILES = 16
EPT = N_CELLS // N_TILES
ECH = 128
ZROWS = 64
CHUNKS = EPT // ECH


def _ln(x, g, b):
    m = jnp.mean(x, axis=-1, keepdims=True)
    v = jnp.mean((x - m) ** 2, axis=-1, keepdims=True)
    return (x - m) * jax.lax.rsqrt(v + 1e-5) * g + b


def _dot(a, b):
    return jnp.dot(a, b, preferred_element_type=jnp.float32)


def _mlp_body(x_ref, w1, b1, w2, b2, w3, b3, o_ref):
    h = jnp.maximum(_dot(x_ref[...], w1[...]) + b1[...], 0.0)
    h = jnp.maximum(_dot(h, w2[...]) + b2[...], 0.0)
    o_ref[...] = _dot(h, w3[...]) + b3[...]


def _mlp_call(x, w1, b1, w2, b2, w3, b3):
    n = x.shape[0]
    grid = n // BLK
    row = pl.BlockSpec((BLK, DIM), lambda i: (i, 0))
    full = pl.BlockSpec((DIM, DIM), lambda i: (0, 0))
    vec = pl.BlockSpec((1, DIM), lambda i: (0, 0))
    return pl.pallas_call(
        _mlp_body,
        grid=(grid,),
        in_specs=[row, full, vec, full, vec, full, vec],
        out_specs=row,
        out_shape=jax.ShapeDtypeStruct((n, DIM), jnp.float32),
    )(x, w1, b1, w2, b2, w3, b3)


def _c_update_body(lc_ref, ch_ref, cc_ref, wih, whh, bsum, g1, bb1, g2, bb2,
                   w1, b1, w2, b2, w3, b3, ch_out, cc_out, cpre_out):
    gates = (_dot(lc_ref[...], wih[...]) + _dot(ch_ref[...], whh[...])
             + bsum[...])
    i = gates[:, 0:DIM]
    f = gates[:, DIM:2 * DIM]
    g = gates[:, 2 * DIM:3 * DIM]
    o = gates[:, 3 * DIM:4 * DIM]
    c2 = jax.nn.sigmoid(f) * cc_ref[...] + jax.nn.sigmoid(i) * jnp.tanh(g)
    h2 = jax.nn.sigmoid(o) * jnp.tanh(c2)
    h2 = _ln(h2, g1[...], bb1[...])
    c2 = _ln(c2, g2[...], bb2[...])
    ch_out[...] = h2
    cc_out[...] = c2
    h = jnp.maximum(_dot(h2, w1[...]) + b1[...], 0.0)
    h = jnp.maximum(_dot(h, w2[...]) + b2[...], 0.0)
    cpre_out[...] = _dot(h, w3[...]) + b3[...]


def _c_update_call(LC, C_h, C_c, wihT, whhT, bsum, g1, b1, g2, b2,
                   w1T, vb1, w2T, vb2, w3T, vb3):
    grid = N_CLAUSES // BLK
    row = pl.BlockSpec((BLK, DIM), lambda i: (i, 0))
    w4 = pl.BlockSpec((DIM, 4 * DIM), lambda i: (0, 0))
    v4 = pl.BlockSpec((1, 4 * DIM), lambda i: (0, 0))
    full = pl.BlockSpec((DIM, DIM), lambda i: (0, 0))
    vec = pl.BlockSpec((1, DIM), lambda i: (0, 0))
    out = jax.ShapeDtypeStruct((N_CLAUSES, DIM), jnp.float32)
    return pl.pallas_call(
        _c_update_body,
        grid=(grid,),
        in_specs=[row, row, row, w4, w4, v4, vec, vec, vec, vec,
                  full, vec, full, vec, full, vec],
        out_specs=(row, row, row),
        out_shape=(out, out, out),
    )(LC, C_h, C_c, wihT, whhT, bsum, g1, b1, g2, b2,
      w1T, vb1, w2T, vb2, w3T, vb3)


def _l_update_body(cl_ref, lh_ref, flip_ref, lcell_ref, wih_a, wih_b, whh,
                   bsum, g1, bb1, g2, bb2, w1, b1, w2, b2, w3, b3,
                   lh_out, lcell_out, lpre_out):
    gates = (_dot(cl_ref[...], wih_a[...]) + _dot(flip_ref[...], wih_b[...])
             + _dot(lh_ref[...], whh[...]) + bsum[...])
    i = gates[:, 0:DIM]
    f = gates[:, DIM:2 * DIM]
    g = gates[:, 2 * DIM:3 * DIM]
    o = gates[:, 3 * DIM:4 * DIM]
    c2 = jax.nn.sigmoid(f) * lcell_ref[...] + jax.nn.sigmoid(i) * jnp.tanh(g)
    h2 = jax.nn.sigmoid(o) * jnp.tanh(c2)
    h2 = _ln(h2, g1[...], bb1[...])
    c2 = _ln(c2, g2[...], bb2[...])
    lh_out[...] = h2
    lcell_out[...] = c2
    h = jnp.maximum(_dot(h2, w1[...]) + b1[...], 0.0)
    h = jnp.maximum(_dot(h, w2[...]) + b2[...], 0.0)
    lpre_out[...] = _dot(h, w3[...]) + b3[...]


def _l_update_call(CL, L_h, L_c, wih_aT, wih_bT, whhT, bsum, g1, b1, g2, b2,
                   w1T, vb1, w2T, vb2, w3T, vb3):
    grid = N_LITS // BLK
    half = grid // 2
    row = pl.BlockSpec((BLK, DIM), lambda i: (i, 0))
    flip = pl.BlockSpec((BLK, DIM), lambda i: ((i + half) % grid, 0))
    full = pl.BlockSpec((DIM, DIM), lambda i: (0, 0))
    w4 = pl.BlockSpec((DIM, 4 * DIM), lambda i: (0, 0))
    v4 = pl.BlockSpec((1, 4 * DIM), lambda i: (0, 0))
    vec = pl.BlockSpec((1, DIM), lambda i: (0, 0))
    out = jax.ShapeDtypeStruct((N_LITS, DIM), jnp.float32)
    return pl.pallas_call(
        _l_update_body,
        grid=(grid,),
        in_specs=[row, row, flip, row, w4, w4, w4, v4, vec, vec, vec, vec,
                  full, vec, full, vec, full, vec],
        out_specs=(row, row, row),
        out_shape=(out, out, out),
    )(CL, L_h, L_h, L_c, wih_aT, wih_bT, whhT, bsum, g1, b1, g2, b2,
      w1T, vb1, w2T, vb2, w3T, vb3)


def _pool_body(lh_ref, pid_ref, o_ref, acc, cnt):
    j = pl.program_id(0)
    nb = pl.num_programs(0)

    @pl.when(j == 0)
    def _():
        acc[...] = jnp.zeros_like(acc)
        cnt[...] = jnp.zeros_like(cnt)

    pid = pid_ref[0, 0, :]
    onehot = (pid[:, None] == lax.broadcasted_iota(
        jnp.int32, (BLK, N_PROBS), 1)).astype(jnp.float32)
    dn = (((0,), (0,)), ((), ()))
    acc[...] += lax.dot_general(onehot, lh_ref[...], dn,
                                preferred_element_type=jnp.float32)
    cnt[...] += lax.dot_general(onehot, jnp.ones((BLK, DIM), jnp.float32), dn,
                                preferred_element_type=jnp.float32)

    @pl.when(j == nb - 1)
    def _():
        o_ref[...] = acc[...] / jnp.maximum(cnt[...], 1.0)


def _pool_call(L_h, pid3):
    grid = N_LITS // BLK
    return pl.pallas_call(
        _pool_body,
        grid=(grid,),
        in_specs=[pl.BlockSpec((BLK, DIM), lambda i: (i, 0)),
                  pl.BlockSpec((1, 1, BLK), lambda i: (i, 0, 0))],
        out_specs=pl.BlockSpec((N_PROBS, DIM), lambda i: (0, 0)),
        out_shape=jax.ShapeDtypeStruct((N_PROBS, DIM), jnp.float32),
        scratch_shapes=[pltpu.VMEM((N_PROBS, DIM), jnp.float32),
                        pltpu.VMEM((N_PROBS, DIM), jnp.float32)],
    )(L_h, pid3)


def _make_segsum(n_src, n_out, n_slabs):
    passes = n_slabs // 2
    mesh = plsc.VectorSubcoreMesh(core_axis_name="c", subcore_axis_name="s")
    ZR = ZROWS
    stripe_z = (SLAB + N_DUMMY) // N_TILES
    stripe_w = SLAB // N_TILES

    @functools.partial(
        pl.kernel, mesh=mesh,
        out_type=jax.ShapeDtypeStruct((n_out, DIM), jnp.float32),
        scratch_types=[
            pltpu.VMEM((ECH,), jnp.int32),
            pltpu.VMEM((ECH,), jnp.int32),
            pltpu.VMEM((ECH,), jnp.int32),
            pltpu.VMEM((ECH, DIM), jnp.float32),
            pltpu.VMEM((ZR, DIM), jnp.float32),
            pltpu.VMEM_SHARED((SLAB + N_DUMMY, DIM), jnp.float32),
            pltpu.SemaphoreType.DMA,
        ])
    def k(table, gidx, didx, zsrc, out, gv, dv, lv, rows, zbuf, acc, sem):
        c = lax.axis_index("c")
        s = lax.axis_index("s")
        pltpu.sync_copy(zsrc, zbuf)
        for p in range(passes):
            base = (2 * p + c) * SLAB
            for r in range(stripe_z // ZR):
                pltpu.sync_copy(zbuf, acc.at[pl.ds(s * stripe_z + r * ZR, ZR)])
            rem = stripe_z % ZR
            if rem:
                pltpu.sync_copy(
                    zbuf.at[pl.ds(0, rem)],
                    acc.at[pl.ds(s * stripe_z + (stripe_z // ZR) * ZR, rem)])
            plsc.subcore_barrier()

            def chunk(kk, _):
                e0 = pl.multiple_of((s * CHUNKS + kk) * ECH, ECH)
                pltpu.sync_copy(gidx.at[pl.ds(e0, ECH)], gv)
                pltpu.sync_copy(didx.at[pl.ds(e0, ECH)], dv)
                ga = pltpu.async_copy(table.at[gv], rows, sem)
                for i in range(8):
                    d = dv[pl.ds(i * 16, 16)]
                    loc = d - base
                    oob = (loc < 0) | (loc >= SLAB)
                    dummy = SLAB + 8 * lax.iota(jnp.int32, 16) + i
                    lv[pl.ds(i * 16, 16)] = jnp.where(oob, dummy, loc)
                ga.wait()
                pltpu.sync_copy(rows, acc.at[lv], add=True)
                return 0

            lax.fori_loop(0, CHUNKS, chunk, 0)
            plsc.subcore_barrier()
            for r in range(stripe_w // ECH):
                pltpu.sync_copy(
                    acc.at[pl.ds(s * stripe_w + r * ECH, ECH)],
                    out.at[pl.ds(base + s * stripe_w + r * ECH, ECH)])
            rem = stripe_w % ECH
            if rem:
                off = (stripe_w // ECH) * ECH
                pltpu.sync_copy(
                    acc.at[pl.ds(s * stripe_w + off, rem)],
                    out.at[pl.ds(base + s * stripe_w + off, rem)])
            plsc.subcore_barrier()

    return k


@functools.lru_cache(maxsize=None)
def _get_segsum(n_src, n_out, n_slabs):
    return _make_segsum(n_src, n_out, n_slabs)


def _segsum_lc(table, gidx, didx, zsrc):
    return _get_segsum(N_LITS, N_CLAUSES, N_CLAUSES // SLAB)(
        table, gidx, didx, zsrc)


def _segsum_cl(table, gidx, didx, zsrc):
    return _get_segsum(N_CLAUSES, N_LITS, N_LITS // SLAB)(
        table, gidx, didx, zsrc)


def kernel(lit_idx, clause_idx, prob_index, params):
    p = params
    lit2d = lit_idx
    cls2d = clause_idx
    zsrc = jnp.zeros((ZROWS, DIM), jnp.float32)

    def tr(w):
        return w.T

    def rv(b):
        return b.reshape(1, -1)

    Lm, Cm = p['L_msg'], p['C_msg']
    LmW = (tr(Lm['W1']), rv(Lm['b1']), tr(Lm['W2']), rv(Lm['b2']),
           tr(Lm['W3']), rv(Lm['b3']))
    CmW = (tr(Cm['W1']), rv(Cm['b1']), tr(Cm['W2']), rv(Cm['b2']),
           tr(Cm['W3']), rv(Cm['b3']))
    Cu, Lu = p['C_up'], p['L_up']
    c_wihT = tr(Cu['Wih'])
    c_whhT = tr(Cu['Whh'])
    c_bsum = rv(Cu['bih'] + Cu['bhh'])
    l_wih_aT = tr(Lu['Wih'][:, :DIM])
    l_wih_bT = tr(Lu['Wih'][:, DIM:])
    l_whhT = tr(Lu['Whh'])
    l_bsum = rv(Lu['bih'] + Lu['bhh'])

    L_h = jnp.broadcast_to(p['L_init_W'][:, 0] + p['L_init_b'], (N_LITS, DIM))
    C_h = jnp.broadcast_to(p['C_init_W'][:, 0] + p['C_init_b'],
                           (N_CLAUSES, DIM))
    L_c = jnp.zeros((N_LITS, DIM), jnp.float32)
    C_c = jnp.zeros((N_CLAUSES, DIM), jnp.float32)

    L_pre = _mlp_call(L_h, *LmW)
    for _ in range(N_ROUNDS):
        LC = _segsum_lc(L_pre, lit2d, cls2d, zsrc)
        C_h, C_c, C_pre = _c_update_call(
            LC, C_h, C_c, c_wihT, c_whhT, c_bsum,
            rv(p['C_lm1_g']), rv(p['C_lm1_b']),
            rv(p['C_lm2_g']), rv(p['C_lm2_b']), *CmW)
        CL = _segsum_cl(C_pre, cls2d, lit2d, zsrc)
        L_h, L_c, L_pre = _l_update_call(
            CL, L_h, L_c, l_wih_aT, l_wih_bT, l_whhT, l_bsum,
            rv(p['L_lm1_g']), rv(p['L_lm1_b']),
            rv(p['L_lm2_g']), rv(p['L_lm2_b']), *LmW)

    pid3 = jnp.concatenate([prob_index, prob_index]).reshape(
        N_LITS // BLK, 1, BLK)
    return _pool_call(L_h, pid3)
